# baseline (speedup 1.0000x reference)
import jax
import jax.numpy as jnp
from jax import lax
from jax.experimental import pallas as pl
from jax.experimental.pallas import tpu as pltpu
from jax.experimental.pallas import tpu_sc as plsc

VOCAB = 100001
FEAT = 64
NFIELD = 7
B, T = 4096, 50
N = B * T
NC, NS = 2, 16
NW = NC * NS
TOK_PER_W = N // NW
CHUNK = 128
NCHUNK = TOK_PER_W // CHUNK


def _sc_body(idx_hbm, *refs):
    tables = refs[:2 * NFIELD]
    re_out, im_out = refs[2 * NFIELD:2 * NFIELD + 2]
    idx_v, rbuf, ibuf, sem_r, sem_i = refs[2 * NFIELD + 2:]

    wid = lax.axis_index("s") * NC + lax.axis_index("c")
    tile_base = wid * TOK_PER_W

    def chunk_body(c, carry):
        base = tile_base + c * CHUNK
        pltpu.sync_copy(idx_hbm.at[:, pl.ds(base, CHUNK)], idx_v)
        for f in range(NFIELD):
            cp_r = pltpu.async_copy(tables[2 * f].at[idx_v.at[f]], rbuf, sem_r)
            cp_i = pltpu.async_copy(tables[2 * f + 1].at[idx_v.at[f]], ibuf, sem_i)
            cp_r.wait()
            pltpu.sync_copy(rbuf, re_out.at[pl.ds(base, CHUNK), pl.ds(f * FEAT, FEAT)])
            cp_i.wait()
            pltpu.sync_copy(ibuf, im_out.at[pl.ds(base, CHUNK), pl.ds(f * FEAT, FEAT)])
        return carry

    lax.fori_loop(0, NCHUNK, chunk_body, 0)


@jax.jit
def _sc_gather(idxT, *tables):
    fn = pl.kernel(
        _sc_body,
        out_type=(
            jax.ShapeDtypeStruct((N, NFIELD * FEAT), jnp.float32),
            jax.ShapeDtypeStruct((N, NFIELD * FEAT), jnp.float32),
        ),
        mesh=plsc.VectorSubcoreMesh(core_axis_name="c", subcore_axis_name="s"),
        scratch_types=[
            pltpu.VMEM((NFIELD, CHUNK), jnp.int32),
            pltpu.VMEM((CHUNK, FEAT), jnp.float32),
            pltpu.VMEM((CHUNK, FEAT), jnp.float32),
            pltpu.SemaphoreType.DMA,
            pltpu.SemaphoreType.DMA,
        ],
        compiler_params=pltpu.CompilerParams(use_tc_tiling_on_sc=False),
    )
    return fn(idxT, *tables)


def kernel(data, yr_real, yr_imag, mt_real, mt_imag, x_real, x_imag,
           y_real, y_imag, m_real, m_imag, d_real, d_imag, t_real, t_imag):
    idxT = data.reshape(N, NFIELD).T
    re, im = _sc_gather(idxT, yr_real, yr_imag, mt_real, mt_imag, x_real,
                        x_imag, y_real, y_imag, m_real, m_imag, d_real,
                        d_imag, t_real, t_imag)
    re3 = re.reshape(B, T, NFIELD * FEAT)
    im3 = im.reshape(B, T, NFIELD * FEAT)
    return lax.complex(re3, im3)


# speedup vs baseline: 1.0003x; 1.0003x over previous
import jax
import jax.numpy as jnp
from jax import lax
from jax.experimental import pallas as pl
from jax.experimental.pallas import tpu as pltpu
from jax.experimental.pallas import tpu_sc as plsc

VOCAB = 100001
FEAT = 64
NFIELD = 7
B, T = 4096, 50
N = B * T
NC, NS = 2, 16
NW = NC * NS
TOK_PER_W = N // NW
CHUNK = 128
NCHUNK = TOK_PER_W // CHUNK


def _sc_body(idx_hbm, *refs):
    tables = refs[:2 * NFIELD]
    re_out, im_out = refs[2 * NFIELD:2 * NFIELD + 2]
    idx_v, rbuf, ibuf, sem_r, sem_i = refs[2 * NFIELD + 2:]

    wid = lax.axis_index("s") * NC + lax.axis_index("c")
    tile_base = wid * TOK_PER_W

    def chunk_body(c, carry):
        base = tile_base + c * CHUNK
        pltpu.sync_copy(idx_hbm.at[:, pl.ds(base, CHUNK)], idx_v)
        for f in range(NFIELD):
            cp_r = pltpu.async_copy(tables[2 * f].at[idx_v.at[f]], rbuf, sem_r)
            cp_i = pltpu.async_copy(tables[2 * f + 1].at[idx_v.at[f]], ibuf, sem_i)
            cp_r.wait()
            pltpu.sync_copy(rbuf, re_out.at[pl.ds(base, CHUNK), pl.ds(f * FEAT, FEAT)])
            cp_i.wait()
            pltpu.sync_copy(ibuf, im_out.at[pl.ds(base, CHUNK), pl.ds(f * FEAT, FEAT)])
        return carry

    lax.fori_loop(0, NCHUNK, chunk_body, 0)


@jax.jit
def _sc_gather(idxT, *tables):
    fn = pl.kernel(
        _sc_body,
        out_type=(
            jax.ShapeDtypeStruct((N, NFIELD * FEAT), jnp.float32),
            jax.ShapeDtypeStruct((N, NFIELD * FEAT), jnp.float32),
        ),
        mesh=plsc.VectorSubcoreMesh(core_axis_name="c", subcore_axis_name="s"),
        scratch_types=[
            pltpu.VMEM((NFIELD, CHUNK), jnp.int32),
            pltpu.VMEM((CHUNK, FEAT), jnp.float32),
            pltpu.VMEM((CHUNK, FEAT), jnp.float32),
            pltpu.SemaphoreType.DMA,
            pltpu.SemaphoreType.DMA,
        ],
        compiler_params=pltpu.CompilerParams(use_tc_tiling_on_sc=False),
    )
    return fn(idxT, *tables)


def kernel(data, yr_real, yr_imag, mt_real, mt_imag, x_real, x_imag,
           y_real, y_imag, m_real, m_imag, d_real, d_imag, t_real, t_imag):
    idxT = data.reshape(N, NFIELD).T
    re, im = _sc_gather(idxT, yr_real, yr_imag, mt_real, mt_imag, x_real,
                        x_imag, y_real, y_imag, m_real, m_imag, d_real,
                        d_imag, t_real, t_imag)
    re3 = re.reshape(B, T, NFIELD * FEAT)
    im3 = im.reshape(B, T, NFIELD * FEAT)
    re3, im3 = lax.optimization_barrier((re3, im3))
    return lax.complex(re3, im3)


# speedup vs baseline: 1.0234x; 1.0231x over previous
import jax
import jax.numpy as jnp
from jax import lax
from jax.experimental import pallas as pl
from jax.experimental.pallas import tpu as pltpu
from jax.experimental.pallas import tpu_sc as plsc

VOCAB = 100001
FEAT = 64
NFIELD = 7
B, T = 4096, 50
N = B * T
NC, NS = 2, 16
NW = NC * NS
TOK_PER_W = N // NW
CHUNK = 320
NCHUNK = TOK_PER_W // CHUNK
NSTREAM = 2 * NFIELD


def _sc_body(idx_hbm, *refs):
    tables = refs[:NSTREAM]
    re_out = refs[NSTREAM]
    im_out = refs[NSTREAM + 1]
    idx_full, gbuf, gsem, osem = refs[NSTREAM + 2:]

    wid = lax.axis_index("s") * NC + lax.axis_index("c")
    tile_base = wid * TOK_PER_W

    pltpu.sync_copy(idx_hbm.at[:, pl.ds(tile_base, TOK_PER_W)], idx_full)

    def chunk_body(c, carry):
        cbase = c * CHUNK
        obase = tile_base + cbase
        gd = [None] * NSTREAM
        od = [None] * NSTREAM

        def ocopy(s):
            f = s // 2
            out = re_out if s % 2 == 0 else im_out
            dst = out.at[pl.ds(obase, CHUNK), pl.ds(f * FEAT, FEAT)]
            return pltpu.async_copy(gbuf.at[s % 2], dst, osem.at[s % 2])

        for s in range(NSTREAM):
            if s >= 2:
                od[s - 2].wait()
            idx_ref = idx_full.at[s // 2, pl.ds(cbase, CHUNK)]
            gd[s] = pltpu.async_copy(
                tables[s].at[idx_ref], gbuf.at[s % 2], gsem.at[s % 2])
            if s >= 1:
                gd[s - 1].wait()
                od[s - 1] = ocopy(s - 1)
        gd[NSTREAM - 1].wait()
        od[NSTREAM - 1] = ocopy(NSTREAM - 1)
        od[NSTREAM - 2].wait()
        od[NSTREAM - 1].wait()
        return carry

    lax.fori_loop(0, NCHUNK, chunk_body, 0)


@jax.jit
def _sc_gather(idxT, *tables):
    fn = pl.kernel(
        _sc_body,
        out_type=(
            jax.ShapeDtypeStruct((N, NFIELD * FEAT), jnp.float32),
            jax.ShapeDtypeStruct((N, NFIELD * FEAT), jnp.float32),
        ),
        mesh=plsc.VectorSubcoreMesh(core_axis_name="c", subcore_axis_name="s"),
        scratch_types=[
            pltpu.VMEM((NFIELD, TOK_PER_W), jnp.int32),
            pltpu.VMEM((2, CHUNK, FEAT), jnp.float32),
            pltpu.SemaphoreType.DMA((2,)),
            pltpu.SemaphoreType.DMA((2,)),
        ],
        compiler_params=pltpu.CompilerParams(use_tc_tiling_on_sc=False),
    )
    return fn(idxT, *tables)


def kernel(data, yr_real, yr_imag, mt_real, mt_imag, x_real, x_imag,
           y_real, y_imag, m_real, m_imag, d_real, d_imag, t_real, t_imag):
    idxT = data.reshape(N, NFIELD).T
    re, im = _sc_gather(idxT, yr_real, yr_imag, mt_real, mt_imag, x_real,
                        x_imag, y_real, y_imag, m_real, m_imag, d_real,
                        d_imag, t_real, t_imag)
    re3 = re.reshape(B, T, NFIELD * FEAT)
    im3 = im.reshape(B, T, NFIELD * FEAT)
    return lax.complex(re3, im3)
